# baseline (speedup 1.0000x reference)
import functools

import jax
import jax.numpy as jnp
from jax import lax
from jax.experimental import pallas as pl
from jax.experimental.pallas import tpu as pltpu
from jax.experimental.pallas import tpu_sc as plsc

_N = 10000
_E = 160000
_D = 256
_NH = _N // 2

_NSUB = 16
_NSEG = 32
_SLICE = 2560
_NSLICE = 64
_EPAD = _NSLICE * _SLICE
_LANES = 16
_CHUNK = 64
_QUAD = 4
_PADE = _QUAD * _CHUNK
_SEGCAP = 2 * _SLICE + _PADE
_SEGROWS = _SEGCAP // _CHUNK
_IDXW = 2 * _CHUNK
_MAXQ = _SEGCAP // _PADE
_AGG_ROWS = 5024
_ZROWS = 320
_OROWS = 312
_SPARE = _NH + 16


def _sc_partition(srcs, dsts):
  mesh = plsc.VectorSubcoreMesh(core_axis_name="c", subcore_axis_name="s")

  @functools.partial(
      pl.kernel,
      out_type=(
          jax.ShapeDtypeStruct((2, _NSEG, _SEGROWS, _CHUNK), jnp.int32),
          jax.ShapeDtypeStruct((2, _NSEG, _SEGROWS, _IDXW), jnp.int32),
          jax.ShapeDtypeStruct((2, _NSEG, _LANES), jnp.int32),
      ),
      mesh=mesh,
      compiler_params=pltpu.CompilerParams(needs_layout_passes=False),
      scratch_types=[
          pltpu.VMEM((_SLICE,), jnp.int32),
          pltpu.VMEM((_SLICE,), jnp.int32),
          [pltpu.VMEM((_SEGROWS + 1, _CHUNK), jnp.int32)
           for _ in range(2)],
          [pltpu.VMEM((_SEGROWS + 1, _IDXW), jnp.int32)
           for _ in range(2)],
          pltpu.VMEM((_LANES,), jnp.int32),
      ],
  )
  def k(srcs_hbm, dsts_hbm, src4_hbm, dst4_hbm, nq_hbm, sslice, dslice,
        scomp, dcomp, nqv):
    c = lax.axis_index("c")
    s = lax.axis_index("s")
    w = s * 2 + c
    lanes = lax.iota(jnp.int32, _LANES)
    dump = _SEGCAP + lanes

    def scat(ref, tgt, val):
      plsc.store_scatter(ref, [tgt // _CHUNK, tgt % _CHUNK], val)

    def scat_d(ref, tgt, val):
      t = tgt // _CHUNK
      col = 2 * (tgt % _CHUNK)
      plsc.store_scatter(ref, [t, col], 2 * val)
      plsc.store_scatter(ref, [t, col + 1], 2 * val + 1)

    def one_slice(q2, offs):
      sl = w * 2 + q2
      pltpu.sync_copy(srcs_hbm.at[sl], sslice)
      pltpu.sync_copy(dsts_hbm.at[sl], dslice)

      def compact(i, offs):
        dv = dslice[pl.ds(i * _LANES, _LANES)]
        sv = sslice[pl.ds(i * _LANES, _LANES)]
        new = []
        for h in range(2):
          d = dv - h * _NH
          keep = (d >= 0) & (d < _NH)
          ki = keep.astype(jnp.int32)
          pos = plsc.cumsum(ki)
          offv = jnp.full((_LANES,), offs[h], jnp.int32)
          tgt = jnp.where(keep, offv + pos - 1, dump)
          scat(scomp[h], tgt, sv)
          scat_d(dcomp[h], tgt, d)
          new.append(offs[h] + plsc.all_reduce_population_count(keep)[0])
        return tuple(new)

      return lax.fori_loop(0, _SLICE // _LANES, compact, offs)

    spare16i = jnp.full((_LANES,), _SPARE, jnp.int32)

    def prefill(r, _):
      for kk in range(_IDXW // _LANES):
        dcomp[0][r, pl.ds(kk * _LANES, _LANES)] = 2 * spare16i
        dcomp[1][r, pl.ds(kk * _LANES, _LANES)] = 2 * spare16i
      return ()

    lax.fori_loop(0, _SEGROWS, prefill, ())

    offs = one_slice(0, (0, 0))
    offs = one_slice(1, offs)

    zero16 = jnp.zeros((_LANES,), jnp.int32)
    spare16 = jnp.full((_LANES,), _SPARE, jnp.int32)
    for h in range(2):
      for kk in range(_PADE // _LANES):
        tgt = offs[h] + kk * _LANES + lanes
        scat(scomp[h], tgt, zero16)
        scat_d(dcomp[h], tgt, spare16)
      nq = offs[h] // _PADE + 1
      nqv[pl.ds(0, _LANES)] = jnp.full((_LANES,), nq, jnp.int32)
      pltpu.sync_copy(nqv, nq_hbm.at[h, w])
      pltpu.sync_copy(scomp[h].at[pl.ds(0, _SEGROWS)], src4_hbm.at[h, w])
      pltpu.sync_copy(dcomp[h].at[pl.ds(0, _SEGROWS)], dst4_hbm.at[h, w])

  return k(srcs, dsts)


def _sc_aggregate(x3, src4, dst4, nq, zrows):
  mesh = plsc.VectorSubcoreMesh(core_axis_name="c", subcore_axis_name="s")

  @functools.partial(
      pl.kernel,
      out_type=jax.ShapeDtypeStruct((2, 2 * _NH, 128), jnp.float32),
      mesh=mesh,
      scratch_types=[
          pltpu.VMEM((_SEGROWS, _CHUNK), jnp.int32),
          pltpu.VMEM((_QUAD, _IDXW), jnp.int32),
          pltpu.VMEM((_LANES,), jnp.int32),
          [pltpu.VMEM((_CHUNK, 2, 128), jnp.float32) for _ in range(2)],
          [pltpu.SemaphoreType.DMA for _ in range(2)],
          pltpu.VMEM_SHARED((2 * _AGG_ROWS, 128), jnp.float32),
      ],
  )
  def k(x_hbm, src4_hbm, dst4_hbm, nq_hbm, z_hbm, out_hbm, src_all, dgrp,
        cntv, bufs, gsems, agg):
    c = lax.axis_index("c")
    s = lax.axis_index("s")

    @pl.when(s < _NSUB - 1)
    def _zmain():
      pltpu.sync_copy(z_hbm, agg.at[pl.ds(s * 2 * _ZROWS, 2 * _ZROWS)])

    @pl.when(s == _NSUB - 1)
    def _ztail():
      rest = 2 * (_AGG_ROWS - 15 * _ZROWS)
      pltpu.sync_copy(z_hbm.at[pl.ds(0, rest)],
                      agg.at[pl.ds(30 * _ZROWS, rest)])

    plsc.subcore_barrier()

    def start_gather(t, b):
      pltpu.async_copy(x_hbm.at[src_all.at[t]], bufs[b], gsems[b])

    def wait_gather(b):
      pltpu.make_async_copy(x_hbm.at[src_all.at[0]], bufs[b],
                            gsems[b]).wait()

    for seg_i in range(2):
      seg = s * 2 + seg_i
      pltpu.sync_copy(nq_hbm.at[c, seg], cntv)
      nq = cntv[pl.ds(0, _LANES)][0]
      pltpu.sync_copy(src4_hbm.at[c, seg], src_all)

      start_gather(0, 0)
      start_gather(1, 1)

      last = nq * _QUAD - 1

      def quad(qd, _):
        pltpu.sync_copy(dst4_hbm.at[c, seg, pl.ds(qd * _QUAD, _QUAD)],
                        dgrp)
        for qq in range(_QUAD):
          t = qd * _QUAD + qq
          b = qq % 2
          wait_gather(b)
          pltpu.sync_copy(bufs[b].reshape(2 * _CHUNK, 128),
                          agg.at[dgrp.at[qq]], add=True)
          start_gather(jnp.minimum(t + 2, last), b)
        return ()

      lax.fori_loop(0, nq, quad, ())
      wait_gather(0)
      wait_gather(1)

    plsc.subcore_barrier()

    pltpu.sync_copy(agg.at[pl.ds(s * 2 * _OROWS, 2 * _OROWS)],
                    out_hbm.at[c, pl.ds(s * 2 * _OROWS, 2 * _OROWS)])

    @pl.when(s == _NSUB - 1)
    def _epilogue():
      tail = _NSUB * 2 * _OROWS
      pltpu.sync_copy(agg.at[pl.ds(tail, 2 * _NH - tail)],
                      out_hbm.at[c, pl.ds(tail, 2 * _NH - tail)])

  return k(x3, src4, dst4, nq, zrows)


def _tc_linear_body(a_ref, w_ref, b_ref, o_ref):
  dn = (((1,), (1,)), ((), ()))
  acc = lax.dot_general(a_ref[...], w_ref[...], dn,
                        preferred_element_type=jnp.float32)
  o_ref[...] = jnp.maximum(acc + b_ref[...], 0.0)


def _tc_linear(agg, w, b2):
  rows = 1000
  grid = _N // rows
  return pl.pallas_call(
      _tc_linear_body,
      grid=(grid,),
      in_specs=[
          pl.BlockSpec((rows, _D), lambda i: (i, 0)),
          pl.BlockSpec((_D, _D), lambda i: (0, 0)),
          pl.BlockSpec((1, _D), lambda i: (0, 0)),
      ],
      out_specs=pl.BlockSpec((rows, _D), lambda i: (i, 0)),
      out_shape=jax.ShapeDtypeStruct((_N, _D), jnp.float32),
  )(agg, w, b2)


def kernel(x, edge_index, W, b):
  src = edge_index[0].astype(jnp.int32)
  dst = edge_index[1].astype(jnp.int32)
  pad = _EPAD - _E
  srcp = jnp.concatenate([src, jnp.zeros((pad,), jnp.int32)])
  dstp = jnp.concatenate([dst, jnp.full((pad,), 2 * _N, jnp.int32)])
  srcs = srcp.reshape(_NSLICE, _SLICE)
  dsts = dstp.reshape(_NSLICE, _SLICE)
  zrows = jnp.zeros((2 * _ZROWS, 128), jnp.float32)

  src4, dst4, nq = _sc_partition(srcs, dsts)
  x3 = x.reshape(_N, 2, 128)
  agg2 = _sc_aggregate(x3, src4, dst4, nq, zrows)
  agg = agg2.reshape(_N, _D)

  b2 = b.reshape(1, _D)
  return _tc_linear(agg, W, b2)


# speedup vs baseline: 1.9758x; 1.9758x over previous
import functools

import jax
import jax.numpy as jnp
from jax import lax
from jax.experimental import pallas as pl
from jax.experimental.pallas import tpu as pltpu
from jax.experimental.pallas import tpu_sc as plsc

_N = 10000
_E = 160000
_D = 256
_DH = 128

_NSUB = 16
_CHUNK = 128
_NCHUNK = 80
_EPT = _NCHUNK * _CHUNK
_EPAD = _EPT * _NSUB
_NBUF = 2
_DGRP = 8
_AGG_ROWS = 10112
_ZROWS = _AGG_ROWS // _NSUB
_OROWS = 624


def _sc_aggregate(xh, srcs, dst3, zrows):
  mesh = plsc.VectorSubcoreMesh(core_axis_name="c", subcore_axis_name="s")

  @functools.partial(
      pl.kernel,
      out_type=jax.ShapeDtypeStruct((2, _N, _DH), jnp.float32),
      mesh=mesh,
      scratch_types=[
          pltpu.VMEM((_NCHUNK, _CHUNK), jnp.int32),
          pltpu.VMEM((_DGRP, _CHUNK), jnp.int32),
          [pltpu.VMEM((_CHUNK, _DH), jnp.float32) for _ in range(_NBUF)],
          [pltpu.SemaphoreType.DMA for _ in range(_NBUF)],
          pltpu.VMEM_SHARED((_AGG_ROWS, _DH), jnp.float32),
      ],
  )
  def k(xh_hbm, srcs_hbm, dst_hbm, z_hbm, out_hbm, src_all, dst8, bufs,
        gsems, agg):
    c = lax.axis_index("c")
    s = lax.axis_index("s")

    pltpu.sync_copy(z_hbm, agg.at[pl.ds(s * _ZROWS, _ZROWS)])
    pltpu.sync_copy(srcs_hbm.at[c, s], src_all)
    plsc.subcore_barrier()

    def start_gather(t, b):
      pltpu.async_copy(xh_hbm.at[src_all.at[t]], bufs[b], gsems[b])

    def wait_gather(b):
      pltpu.make_async_copy(xh_hbm.at[src_all.at[0]], bufs[b],
                            gsems[b]).wait()

    for b in range(_NBUF):
      start_gather(b, b)

    def group(g, static_tail):
      pltpu.sync_copy(dst_hbm.at[s, pl.ds(g * _DGRP, _DGRP)], dst8)
      for q in range(_DGRP):
        t = g * _DGRP + q
        b = q % _NBUF
        wait_gather(b)
        pltpu.sync_copy(bufs[b], agg.at[dst8.at[q]], add=True)
        if static_tail:
          if q < _DGRP - _NBUF:
            start_gather(t + _NBUF, b)
        else:
          start_gather(t + _NBUF, b)

    def body(g, _):
      group(g, False)
      return ()

    lax.fori_loop(0, _NCHUNK // _DGRP - 1, body, ())
    group(_NCHUNK // _DGRP - 1, True)

    plsc.subcore_barrier()

    pltpu.sync_copy(agg.at[pl.ds(s * _OROWS, _OROWS)],
                    out_hbm.at[c, pl.ds(s * _OROWS, _OROWS)])

    @pl.when(s == _NSUB - 1)
    def _epilogue():
      tail = _NSUB * _OROWS
      pltpu.sync_copy(agg.at[pl.ds(tail, _N - tail)],
                      out_hbm.at[c, pl.ds(tail, _N - tail)])

  return k(xh, srcs, dst3, zrows)


def _tc_linear_body(a0_ref, a1_ref, w0_ref, w1_ref, b_ref, o_ref):
  dn = (((1,), (1,)), ((), ()))
  acc = lax.dot_general(a0_ref[0], w0_ref[...], dn,
                        preferred_element_type=jnp.float32)
  acc += lax.dot_general(a1_ref[0], w1_ref[...], dn,
                         preferred_element_type=jnp.float32)
  o_ref[...] = jnp.maximum(acc + b_ref[...], 0.0)


def _tc_linear(agg2, w0, w1, b2):
  rows = 1000
  grid = _N // rows
  return pl.pallas_call(
      _tc_linear_body,
      grid=(grid,),
      in_specs=[
          pl.BlockSpec((1, rows, _DH), lambda i: (0, i, 0)),
          pl.BlockSpec((1, rows, _DH), lambda i: (1, i, 0)),
          pl.BlockSpec((_D, _DH), lambda i: (0, 0)),
          pl.BlockSpec((_D, _DH), lambda i: (0, 0)),
          pl.BlockSpec((1, _D), lambda i: (0, 0)),
      ],
      out_specs=pl.BlockSpec((rows, _D), lambda i: (i, 0)),
      out_shape=jax.ShapeDtypeStruct((_N, _D), jnp.float32),
  )(agg2, agg2, w0, w1, b2)


def kernel(x, edge_index, W, b):
  src = edge_index[0].astype(jnp.int32)
  dst = edge_index[1].astype(jnp.int32)
  pad = _EPAD - _E
  srcp = jnp.concatenate([src, jnp.zeros((pad,), jnp.int32)])
  dstp = jnp.concatenate([dst, jnp.full((pad,), _N, jnp.int32)])
  srcs = jnp.concatenate([srcp, srcp + _N]).reshape(
      2, _NSUB, _NCHUNK, _CHUNK)
  dst3 = dstp.reshape(_NSUB, _NCHUNK, _CHUNK)
  xh = jnp.concatenate([x[:, :_DH], x[:, _DH:]], axis=0)
  zrows = jnp.zeros((_ZROWS, _DH), jnp.float32)

  agg2 = _sc_aggregate(xh, srcs, dst3, zrows)

  w0 = W[:, :_DH]
  w1 = W[:, _DH:]
  b2 = b.reshape(1, _D)
  return _tc_linear(agg2, w0, w1, b2)
